# baseline (speedup 1.0000x reference)
import jax
import jax.numpy as jnp
from jax import lax
from jax.experimental import pallas as pl
from jax.experimental.pallas import tpu as pltpu
from jax.experimental.pallas import tpu_sc as plsc

N_NODES = 10000
N_EDGES = 160000
D_FEAT = 128
T = 16
S = 16
G = 32
PAD_H = 1000.0

NW = 32
N_PAD = 10240
E_PAD = 163840
NODES_PER_TILE = N_PAD // NW
EDGES_PER_TILE = E_PAD // NW
CHUNK = 640
N_CHUNKS = EDGES_PER_TILE // CHUNK
SUB = 128
N_SUB = CHUNK // SUB

GT = G * T
DSIZE = 17 * GT
ROW_BLK = 1024

_C0 = 500.0 * 2.0 / 15.0


def _tc_prep_body(x_ref, w_ref, b_ref, v_ref, nhg_ref, nh_ref):
    xw = x_ref[...] * w_ref[0, 0, :][:, None]
    nh = lax.dot(xw, v_ref[...], precision=lax.Precision.HIGHEST,
                 preferred_element_type=jnp.float32)
    row = pl.program_id(0) * ROW_BLK + lax.broadcasted_iota(
        jnp.int32, (ROW_BLK, 1), 0)
    nh = jnp.where(row < N_NODES, nh, PAD_H)
    nh_ref[...] = nh
    gcol = (b_ref[0, 0, :].astype(jnp.int32)[:, None] * T
            + lax.broadcasted_iota(jnp.int32, (1, T), 1)).astype(jnp.float32)
    nhg_ref[...] = jnp.concatenate([nh, gcol], axis=1)


def _tc_prep(x_pad, nw3, b3, v):
    grid = N_PAD // ROW_BLK
    return pl.pallas_call(
        _tc_prep_body,
        grid=(grid,),
        in_specs=[
            pl.BlockSpec((ROW_BLK, D_FEAT), lambda i: (i, 0)),
            pl.BlockSpec((1, 1, ROW_BLK), lambda i: (i, 0, 0)),
            pl.BlockSpec((1, 1, ROW_BLK), lambda i: (i, 0, 0)),
            pl.BlockSpec((D_FEAT, T), lambda i: (0, 0)),
        ],
        out_specs=[
            pl.BlockSpec((ROW_BLK, 2 * T), lambda i: (i, 0)),
            pl.BlockSpec((ROW_BLK, T), lambda i: (i, 0)),
        ],
        out_shape=[
            jax.ShapeDtypeStruct((N_PAD, 2 * T), jnp.float32),
            jax.ShapeDtypeStruct((N_PAD, T), jnp.float32),
        ],
    )(x_pad, nw3, b3, v)


def _accum(h, gbi, sign, d_ref):
    w = h * 7.5 + 8.0
    ri = jnp.clip(w.astype(jnp.int32), 0, S - 1)
    z = (h * 500.0 + 500.0) - ri.astype(jnp.float32) * _C0
    s1 = sign / (1.0 + jnp.exp(z))
    idx = ri * GT + gbi
    plsc.addupdate_scatter(d_ref, [idx], s1)
    plsc.addupdate_scatter(d_ref, [idx + GT], sign - s1)


def _sc_body(nhg_hbm, nh_hbm, src_hbm, dst_hbm, ew_hbm, out_hbm,
             d_v, sr0, sr1, dr0, dr1, noderows, si0, si1, di0, di1,
             ew0, ew1, sem_n, sem_i0, sem_i1, sem_r0, sem_r1):
    wid = lax.axis_index("s") * 2 + lax.axis_index("c")
    sr, dr, si, di, ewb = [sr0, sr1], [dr0, dr1], [si0, si1], [di0, di1], \
        [ew0, ew1]
    sem_r = [sem_r0, sem_r1]

    def idx_copies(c, b, sem):
        ebase = pl.multiple_of((wid * N_CHUNKS + c) * CHUNK, CHUNK)
        return [
            pltpu.make_async_copy(
                src_hbm.at[pl.ds(ebase, CHUNK)], si[b], sem),
            pltpu.make_async_copy(
                dst_hbm.at[pl.ds(ebase, CHUNK)], di[b], sem),
            pltpu.make_async_copy(
                ew_hbm.at[pl.ds(ebase, CHUNK)], ewb[b], sem),
        ]

    def row_copies(b):
        cps = []
        for k in range(N_SUB):
            cps.append(pltpu.make_async_copy(
                nhg_hbm.at[si[b].at[pl.ds(k * SUB, SUB)]],
                sr[b].at[pl.ds(k * SUB, SUB), :], sem_r[b]))
            cps.append(pltpu.make_async_copy(
                nh_hbm.at[di[b].at[pl.ds(k * SUB, SUB)]],
                dr[b].at[pl.ds(k * SUB, SUB), :], sem_r[b]))
        return cps

    def start_all(cps):
        for cp in cps:
            cp.start()

    def wait_all(cps):
        for cp in cps:
            cp.wait()

    def compute_chunk(b):
        @plsc.parallel_loop(0, CHUNK // 16)
        def _grp(grp):
            g16 = grp * 16
            ew16 = ewb[b][pl.ds(g16, 16)]
            for j in range(16):
                e = g16 + j
                hs = sr[b][e, pl.ds(0, T)]
                gbi = sr[b][e, pl.ds(T, T)].astype(jnp.int32)
                hd = dr[b][e, pl.ds(0, T)]
                ewj = jnp.take_along_axis(
                    ew16, jnp.full((16,), j, jnp.int32), axis=0)
                h = jnp.maximum(hs, hd) * ewj
                _accum(h, gbi, -1.0, d_v)

    cps = idx_copies(0, 0, sem_i0)
    start_all(cps)
    wait_all(cps)
    start_all(row_copies(0))
    start_all(idx_copies(1, 1, sem_i1))
    nbase = pl.multiple_of(wid * NODES_PER_TILE, NODES_PER_TILE)
    node_cp = pltpu.make_async_copy(
        nhg_hbm.at[pl.ds(nbase, NODES_PER_TILE), :], noderows, sem_n)
    node_cp.start()

    @plsc.parallel_loop(0, DSIZE // 16, unroll=4)
    def _zero(j):
        d_v[pl.ds(j * 16, 16)] = jnp.zeros((16,), jnp.float32)

    node_cp.wait()

    @plsc.parallel_loop(0, NODES_PER_TILE, unroll=2)
    def _node(n):
        h = noderows[n, pl.ds(0, T)]
        gbi = noderows[n, pl.ds(T, T)].astype(jnp.int32)
        _accum(h, gbi, 1.0, d_v)

    def pair_body(c2, _):
        p0 = 2 * c2
        not_last = c2 < N_CHUNKS // 2 - 1
        wait_all(idx_copies(p0 + 1, 1, sem_i1))
        start_all(row_copies(1))
        wait_all(row_copies(0))
        compute_chunk(0)

        @pl.when(not_last)
        def _():
            start_all(idx_copies(p0 + 2, 0, sem_i0))
            wait_all(idx_copies(p0 + 2, 0, sem_i0))
            start_all(row_copies(0))

        wait_all(row_copies(1))
        compute_chunk(1)

        @pl.when(not_last)
        def _():
            start_all(idx_copies(p0 + 3, 1, sem_i1))

        return 0

    lax.fori_loop(0, N_CHUNKS // 2, pair_body, 0)

    obase = pl.multiple_of(wid * DSIZE, DSIZE)
    pltpu.sync_copy(d_v, out_hbm.at[pl.ds(obase, DSIZE)])


def _sc_call(nhg, nh16, src2, dst2, ewp):
    mesh = plsc.VectorSubcoreMesh(core_axis_name="c", subcore_axis_name="s")
    f = pl.kernel(
        _sc_body,
        out_type=jax.ShapeDtypeStruct((NW * DSIZE,), jnp.float32),
        mesh=mesh,
        compiler_params=pltpu.CompilerParams(
            needs_layout_passes=False, use_tc_tiling_on_sc=False),
        scratch_types=[
            pltpu.VMEM((DSIZE,), jnp.float32),
            pltpu.VMEM((CHUNK, 2 * T), jnp.float32),
            pltpu.VMEM((CHUNK, 2 * T), jnp.float32),
            pltpu.VMEM((CHUNK, T), jnp.float32),
            pltpu.VMEM((CHUNK, T), jnp.float32),
            pltpu.VMEM((NODES_PER_TILE, 2 * T), jnp.float32),
            pltpu.VMEM((CHUNK,), jnp.int32),
            pltpu.VMEM((CHUNK,), jnp.int32),
            pltpu.VMEM((CHUNK,), jnp.int32),
            pltpu.VMEM((CHUNK,), jnp.int32),
            pltpu.VMEM((CHUNK,), jnp.float32),
            pltpu.VMEM((CHUNK,), jnp.float32),
            pltpu.SemaphoreType.DMA,
            pltpu.SemaphoreType.DMA,
            pltpu.SemaphoreType.DMA,
            pltpu.SemaphoreType.DMA,
            pltpu.SemaphoreType.DMA,
        ],
    )
    return f(nhg, nh16, src2, dst2, ewp)


def _tc_fin_body(p_ref, out_ref):
    s0 = jnp.sum(p_ref[...], axis=0)
    acc = jnp.zeros((G, T), jnp.float32)
    for s in range(S):
        acc = acc + s0[s]
        out_ref[:, s:s + 1, :] = acc[:, None, :]


def _tc_fin(partials):
    return pl.pallas_call(
        _tc_fin_body,
        out_shape=jax.ShapeDtypeStruct((G, S, T), jnp.float32),
    )(partials)


def kernel(x, node_weights, edge_index, edge_weights, batch, v, lin):
    del lin
    f32, i32 = jnp.float32, jnp.int32
    x_pad = jnp.concatenate(
        [x, jnp.zeros((N_PAD - N_NODES, D_FEAT), f32)], axis=0)
    nw3 = jnp.concatenate(
        [node_weights, jnp.zeros((N_PAD - N_NODES,), f32)]).reshape(
            N_PAD // ROW_BLK, 1, ROW_BLK)
    b3 = jnp.concatenate(
        [batch, jnp.zeros((N_PAD - N_NODES,), i32)]).reshape(
            N_PAD // ROW_BLK, 1, ROW_BLK)

    epad = E_PAD - N_EDGES
    src2 = jnp.concatenate(
        [edge_index[0], jnp.full((epad,), N_NODES, i32)])
    dst2 = jnp.concatenate(
        [edge_index[1], jnp.full((epad,), N_NODES, i32)])
    ewp = jnp.concatenate([edge_weights, jnp.ones((epad,), f32)])

    nhg, nh16 = _tc_prep(x_pad, nw3, b3, v)
    partials = _sc_call(nhg, nh16, src2, dst2, ewp)
    out = _tc_fin(partials.reshape(NW, 17, G, T))
    return out
